# baseline (speedup 1.0000x reference)
import functools

import jax
import jax.numpy as jnp
from jax import lax
from jax.experimental import pallas as pl
from jax.experimental.pallas import tpu as pltpu
from jax.experimental.pallas import tpu_sc as plsc

D_MODEL_ = 2048
N_FEAT_ = 32768
K_ = 64
BATCH_ = 1024

F_TILE = 512
NW = 32
ROWS_W = BATCH_ // NW
CAP = 256
NV = CAP // 16
CAPL = 32
NGRP = N_FEAT_ // 16
GCAP = 256
GBUF = GCAP + 16
import numpy as _np
M_SIGN = _np.int32(-2**31)
S_MIN = _np.int32(-2**31)
I_BIG = _np.int32(2**30)


def _encode_body(x_ref, w_ref, be_ref, bd_ref, out_ref, gmax_ref):
    xt = x_ref[...] - bd_ref[...]
    acts = lax.dot_general(
        xt, w_ref[...],
        dimension_numbers=(((1,), (1,)), ((), ())),
        preferred_element_type=jnp.float32,
    ) + be_ref[...]
    out_ref[...] = acts
    B = acts.shape[0]
    GPT = F_TILE // 16
    gmax_ref[0] = acts.reshape(B, GPT, 16).max(axis=2)


def _encode(x, W_enc, b_enc, b_dec):
    B, D = x.shape
    F = W_enc.shape[0]
    grid = (F // F_TILE,)
    GPT = F_TILE // 16
    out_shapes = (
        jax.ShapeDtypeStruct((B, F), jnp.float32),
        jax.ShapeDtypeStruct((F // F_TILE, B, GPT), jnp.float32),
    )
    pre_acts, gmax3 = pl.pallas_call(
        _encode_body,
        grid=grid,
        in_specs=[
            pl.BlockSpec((B, D), lambda j: (0, 0)),
            pl.BlockSpec((F_TILE, D), lambda j: (j, 0)),
            pl.BlockSpec((1, F_TILE), lambda j: (0, j)),
            pl.BlockSpec((1, D), lambda j: (0, 0)),
        ],
        out_specs=(
            pl.BlockSpec((B, F_TILE), lambda j: (0, j)),
            pl.BlockSpec((1, B, GPT), lambda j: (j, 0, 0)),
        ),
        out_shape=out_shapes,
    )(x, W_enc, b_enc.reshape(1, F), b_dec.reshape(1, D))
    gmax = jnp.transpose(gmax3, (1, 0, 2)).reshape(B, F // 16)
    return pre_acts, gmax


def _vperm(x, idx):
    dnums = lax.GatherDimensionNumbers(
        offset_dims=(), collapsed_slice_dims=(0,), start_index_map=(0,))
    return lax.gather(x, idx[:, None], dnums, slice_sizes=(1,),
                      mode=lax.GatherScatterMode.PROMISE_IN_BOUNDS)


def _f2s(b):
    return jnp.where(b < 0, jnp.bitwise_xor(jnp.invert(b), M_SIGN), b)


def _s2b(s):
    return jnp.where(s < 0, jnp.bitwise_xor(jnp.invert(s), M_SIGN), s)


def _topk_body(pre_hbm, gmax_hbm, vals_hbm, idx_hbm,
               rowbuf0, rowbuf1, gbuf0, gbuf1, pbuf, lanebuf, gidbuf, candbuf,
               skeys, sidx, ovals, oidx, sem_a, sem_b, sem_c, sem_d):
    wid = lax.axis_index("s") * 2 + lax.axis_index("c")
    base = wid * ROWS_W
    lane = lax.iota(jnp.int32, 16)
    lane0 = lane == 0

    for i in range(GBUF // 16):
        gidbuf[pl.ds(i * 16, 16)] = jnp.zeros(16, jnp.int32)

    pltpu.make_async_copy(pre_hbm.at[base], rowbuf0, sem_a).start()
    pltpu.make_async_copy(gmax_hbm.at[base], gbuf0, sem_c).start()

    def prefix16(cnt):
        tot = cnt
        for sh in (1, 2, 4, 8):
            tot = tot + _vperm(tot, jnp.bitwise_xor(lane, sh))
        incl = cnt
        for sh in (1, 2, 4, 8):
            shifted = _vperm(incl, jnp.maximum(lane - sh, 0))
            incl = incl + jnp.where(lane >= sh, shifted, 0)
        return incl - cnt, tot

    def process(row_ref, gmax_ref, rl):
        def prep(i, _):
            v = gmax_ref[pl.ds(i * 16, 16)]
            s = _f2s(lax.bitcast_convert_type(v, jnp.int32))
            pbuf[pl.ds(i * 16, 16)] = (s >> 20) + 2048
            return 0
        lax.fori_loop(0, NGRP // 16, prep, 0, unroll=8)

        def probe(it, p):
            tryp = p + jnp.full((16,), 1 << (11 - it), jnp.int32)
            acc = jnp.zeros(16, jnp.int32)
            for i in range(NGRP // 16):
                m = pbuf[pl.ds(i * 16, 16)] >= tryp
                acc = acc + jnp.where(m, 1, 0)
            for sh in (1, 2, 4, 8):
                acc = acc + _vperm(acc, jnp.bitwise_xor(lane, sh))
            return jnp.where(acc >= K_, tryp, p)
        p = lax.fori_loop(0, 12, probe, jnp.zeros(16, jnp.int32))
        st = (p - 2048) << 20
        tf = lax.bitcast_convert_type(_s2b(st), jnp.float32)

        lbase0 = lane * CAPL

        def gscan(i, carry):
            gl, gidv = carry
            m = pbuf[pl.ds(i * 16, 16)] >= p
            pos = lbase0 + jnp.minimum(gl, CAPL - 1)
            plsc.store_scatter(lanebuf, [pos], gidv, mask=m)
            return gl + jnp.where(m, 1, 0), gidv + 16
        gl, _ = lax.fori_loop(0, NGRP // 16, gscan,
                              (jnp.zeros(16, jnp.int32), lane), unroll=4)
        gl = jnp.minimum(gl, CAPL)
        glbase, gtot = prefix16(gl)
        ng = jnp.minimum(gtot, GCAP)

        for j in range(2 * 16):
            l, half = j // 2, j % 2
            cj = lanebuf[pl.ds(j * 16, 16)]
            lb = _vperm(glbase, jnp.full((16,), l, jnp.int32))
            cl = _vperm(gl, jnp.full((16,), l, jnp.int32))
            slot = lane + 16 * half
            valid = slot < cl
            pos = jnp.minimum(lb + slot, GCAP + 15)
            plsc.store_scatter(gidbuf, [pos], cj, mask=valid)

        def egroup(vi, loff):
            gvec = gidbuf[pl.ds(vi * 16, 16)]
            for k in range(16):
                gv = _vperm(gvec, jnp.full((16,), k, jnp.int32))
                idxv = gv * 16 + lane
                v = plsc.load_gather(row_ref, [idxv])
                sfull = jnp.full((16,), vi * 16 + k, jnp.int32)
                m = (v >= tf) & (sfull < ng)
                pos = lbase0 + jnp.minimum(loff, CAPL - 1)
                plsc.store_scatter(lanebuf, [pos], idxv, mask=m)
                loff = loff + jnp.where(m, 1, 0)
            return loff
        loff = lax.fori_loop(0, GCAP // 16, egroup, jnp.zeros(16, jnp.int32))
        loff = jnp.minimum(loff, CAPL)

        lbase, ps = prefix16(loff)
        cvalid = jnp.minimum(ps, CAP)

        for j in range(2 * 16):
            l, half = j // 2, j % 2
            cj = lanebuf[pl.ds(j * 16, 16)]
            lb = _vperm(lbase, jnp.full((16,), l, jnp.int32))
            cl = _vperm(loff, jnp.full((16,), l, jnp.int32))
            slot = lane + 16 * half
            valid = slot < cl
            pos = jnp.minimum(lb + slot, CAP + 15)
            plsc.store_scatter(candbuf, [pos], cj, mask=valid)
        for i in range(NV):
            ci = candbuf[pl.ds(i * 16, 16)]
            valid = (lane + i * 16) < cvalid
            ci = jnp.where(valid, ci, 0)
            v = plsc.load_gather(row_ref, [ci])
            s = _f2s(lax.bitcast_convert_type(v, jnp.int32))
            s = jnp.where(valid, s, S_MIN)
            ci = jnp.where(valid, ci, I_BIG)
            sk, sv = plsc.sort_key_val(s, ci, descending=True)
            skeys[pl.ds(i * 17, 16)] = sk
            sidx[pl.ds(i * 17, 16)] = sv
            skeys[pl.ds(i * 17 + 16, 16)] = jnp.full((16,), S_MIN, jnp.int32)
            sidx[pl.ds(i * 17 + 16, 16)] = jnp.full((16,), I_BIG, jnp.int32)

        rlv = jnp.full((16,), rl, jnp.int32)
        zlane = jnp.zeros(16, jnp.int32)
        def pick(t, hp):
            hs = plsc.load_gather(skeys, [hp])
            hi = plsc.load_gather(sidx, [hp])
            ss, si = plsc.sort_key_val(hs, hi, descending=True)
            mx = _vperm(ss, zlane)
            iw = _vperm(si, zlane)
            win = (hs == mx) & (hi == iw)
            hp = hp + jnp.where(win, 1, 0)
            fv = jnp.maximum(lax.bitcast_convert_type(_s2b(mx), jnp.float32), 0.0)
            posv = jnp.full((16,), t, jnp.int32)
            plsc.store_scatter(ovals, [rlv, posv], fv, mask=lane0)
            plsc.store_scatter(oidx, [rlv, posv], iw, mask=lane0)
            return hp
        lax.fori_loop(0, K_, pick, lane * 17)

    def outer(k, _):
        r0 = base + 2 * k
        pltpu.make_async_copy(pre_hbm.at[r0 + 1], rowbuf1, sem_b).start()
        pltpu.make_async_copy(gmax_hbm.at[r0 + 1], gbuf1, sem_d).start()
        pltpu.make_async_copy(pre_hbm.at[r0], rowbuf0, sem_a).wait()
        pltpu.make_async_copy(gmax_hbm.at[r0], gbuf0, sem_c).wait()
        process(rowbuf0, gbuf0, 2 * k)

        @pl.when(k < ROWS_W // 2 - 1)
        def _n0():
            pltpu.make_async_copy(pre_hbm.at[r0 + 2], rowbuf0, sem_a).start()
            pltpu.make_async_copy(gmax_hbm.at[r0 + 2], gbuf0, sem_c).start()
        pltpu.make_async_copy(pre_hbm.at[r0 + 1], rowbuf1, sem_b).wait()
        pltpu.make_async_copy(gmax_hbm.at[r0 + 1], gbuf1, sem_d).wait()
        process(rowbuf1, gbuf1, 2 * k + 1)

        @pl.when(k < ROWS_W // 2 - 1)
        def _n1():
            pltpu.make_async_copy(pre_hbm.at[r0 + 3], rowbuf1, sem_b).start()
            pltpu.make_async_copy(gmax_hbm.at[r0 + 3], gbuf1, sem_d).start()
        return 0
    lax.fori_loop(0, ROWS_W // 2, outer, 0)

    pltpu.sync_copy(ovals, vals_hbm.at[pl.ds(base, ROWS_W)])
    pltpu.sync_copy(oidx, idx_hbm.at[pl.ds(base, ROWS_W)])


def _topk_sc(pre_acts, cmax):
    B = pre_acts.shape[0]
    kfn = pl.kernel(
        _topk_body,
        out_type=(
            jax.ShapeDtypeStruct((B, K_), jnp.float32),
            jax.ShapeDtypeStruct((B, K_), jnp.int32),
        ),
        mesh=plsc.VectorSubcoreMesh(core_axis_name="c", subcore_axis_name="s"),
        compiler_params=pltpu.CompilerParams(needs_layout_passes=False),
        scratch_types=[
            pltpu.VMEM((N_FEAT_,), jnp.float32),
            pltpu.VMEM((N_FEAT_,), jnp.float32),
            pltpu.VMEM((NGRP,), jnp.float32),
            pltpu.VMEM((NGRP,), jnp.float32),
            pltpu.VMEM((NGRP,), jnp.int32),
            pltpu.VMEM((16 * CAPL,), jnp.int32),
            pltpu.VMEM((GBUF,), jnp.int32),
            pltpu.VMEM((CAP + 32,), jnp.int32),
            pltpu.VMEM((NV * 17 + 16,), jnp.int32),
            pltpu.VMEM((NV * 17 + 16,), jnp.int32),
            pltpu.VMEM((ROWS_W, K_), jnp.float32),
            pltpu.VMEM((ROWS_W, K_), jnp.int32),
            pltpu.SemaphoreType.DMA,
            pltpu.SemaphoreType.DMA,
            pltpu.SemaphoreType.DMA,
            pltpu.SemaphoreType.DMA,
        ],
    )
    return kfn(pre_acts, cmax)


SEGS = D_MODEL_ // 256


def _decode_body(wdec_hbm, vals_hbm, idx_hbm, x_hbm, bd_hbm,
                 xhat_hbm, part_hbm,
                 idxb, valb, bdec, acc,
                 g0, g1, xb0, xb1, xh0, xh1, lsbuf,
                 sem_g0, sem_g1, sem_x0, sem_x1, sem_o0, sem_o1):
    wid = lax.axis_index("s") * 2 + lax.axis_index("c")
    base = wid * ROWS_W

    pltpu.sync_copy(idx_hbm.at[pl.ds(base, ROWS_W)], idxb)
    pltpu.sync_copy(vals_hbm.at[pl.ds(base, ROWS_W)], valb)
    pltpu.sync_copy(bd_hbm, bdec)
    pltpu.make_async_copy(x_hbm.at[base], xb0, sem_x0).start()

    def gslice(r, kc):
        return idxb.at[r, pl.ds(kc * 16, 16)]

    def process(r, rl, xb, xh, sem_x, sem_o, lsum):
        pltpu.make_async_copy(wdec_hbm.at[gslice(rl, 0)], g0, sem_g0).start()

        for kc in range(4):
            gb, sem = (g0, sem_g0) if kc % 2 == 0 else (g1, sem_g1)
            nb, nsem = (g1, sem_g1) if kc % 2 == 0 else (g0, sem_g0)
            if kc < 3:
                pltpu.make_async_copy(wdec_hbm.at[gslice(rl, kc + 1)],
                                      nb, nsem).start()
            pltpu.make_async_copy(wdec_hbm.at[gslice(rl, kc)], gb, sem).wait()
            vals16 = valb[rl, pl.ds(kc * 16, 16)]

            def seg_body(sg, _):
                sb = sg * 256
                if kc == 0:
                    a = [jnp.zeros(16, jnp.float32) for _ in range(16)]
                else:
                    a = [acc[pl.ds(sb + j * 16, 16)] for j in range(16)]
                for k in range(16):
                    vv = jnp.full((16,), vals16[k], jnp.float32)
                    for j in range(16):
                        a[j] = a[j] + vv * gb[k, pl.ds(sb + j * 16, 16)]
                for j in range(16):
                    acc[pl.ds(sb + j * 16, 16)] = a[j]
                return _
            lax.fori_loop(0, SEGS, seg_body, 0)

        pltpu.make_async_copy(x_hbm.at[r], xb, sem_x).wait()

        def fin_body(i, ls):
            sl = pl.ds(i * 16, 16)
            xh_v = acc[sl] + bdec[sl]
            xh[sl] = xh_v
            d = xh_v - xb[sl]
            return ls + d * d
        lsum = lax.fori_loop(0, D_MODEL_ // 16, fin_body, lsum, unroll=8)
        pltpu.make_async_copy(xh, xhat_hbm.at[r], sem_o).start()
        return lsum

    def outer(k, lsum):
        r0 = base + 2 * k

        @pl.when(k == 0)
        def _p1():
            pltpu.make_async_copy(x_hbm.at[r0 + 1], xb1, sem_x1).start()
        lsum = process(r0, 2 * k, xb0, xh0, sem_x0, sem_o0, lsum)

        @pl.when(k < ROWS_W // 2 - 1)
        def _n0():
            pltpu.make_async_copy(x_hbm.at[r0 + 2], xb0, sem_x0).start()
        lsum = process(r0 + 1, 2 * k + 1, xb1, xh1, sem_x1, sem_o1, lsum)

        @pl.when(k < ROWS_W // 2 - 1)
        def _n1():
            pltpu.make_async_copy(x_hbm.at[r0 + 3], xb1, sem_x1).start()
        pltpu.make_async_copy(xh0, xhat_hbm.at[r0], sem_o0).wait()
        pltpu.make_async_copy(xh1, xhat_hbm.at[r0 + 1], sem_o1).wait()
        return lsum

    lsum = lax.fori_loop(0, ROWS_W // 2, outer, jnp.zeros(16, jnp.float32))
    lsbuf[...] = lsum
    pltpu.sync_copy(lsbuf, part_hbm.at[wid])


def _decode_sc(W_dec, topk_vals, topk_idx, x, b_dec):
    B = x.shape[0]
    D = x.shape[1]
    kfn = pl.kernel(
        _decode_body,
        out_type=(
            jax.ShapeDtypeStruct((B, D), jnp.float32),
            jax.ShapeDtypeStruct((NW, 16), jnp.float32),
        ),
        mesh=plsc.VectorSubcoreMesh(core_axis_name="c", subcore_axis_name="s"),
        compiler_params=pltpu.CompilerParams(needs_layout_passes=False),
        scratch_types=[
            pltpu.VMEM((ROWS_W, K_), jnp.int32),
            pltpu.VMEM((ROWS_W, K_), jnp.float32),
            pltpu.VMEM((D_MODEL_,), jnp.float32),
            pltpu.VMEM((D_MODEL_,), jnp.float32),
            pltpu.VMEM((16, D_MODEL_), jnp.float32),
            pltpu.VMEM((16, D_MODEL_), jnp.float32),
            pltpu.VMEM((D_MODEL_,), jnp.float32),
            pltpu.VMEM((D_MODEL_,), jnp.float32),
            pltpu.VMEM((D_MODEL_,), jnp.float32),
            pltpu.VMEM((D_MODEL_,), jnp.float32),
            pltpu.VMEM((16,), jnp.float32),
            pltpu.SemaphoreType.DMA,
            pltpu.SemaphoreType.DMA,
            pltpu.SemaphoreType.DMA,
            pltpu.SemaphoreType.DMA,
            pltpu.SemaphoreType.DMA,
            pltpu.SemaphoreType.DMA,
        ],
    )
    return kfn(W_dec, topk_vals, topk_idx, x, b_dec)


def kernel(x, W_enc, b_enc, W_dec, b_dec):
    B = x.shape[0]
    F = W_enc.shape[0]
    pre_acts, cmax = _encode(x, W_enc, b_enc, b_dec)
    topk_vals, topk_idx = _topk_sc(pre_acts, cmax)
    x_hat, partials = _decode_sc(W_dec, topk_vals, topk_idx, x, b_dec)
    recon_loss = jnp.sum(partials) / (x.shape[0] * x.shape[1])
    aux_loss = jnp.float32(0.0)
    loss = recon_loss
    return (x_hat, topk_vals, topk_idx, recon_loss, aux_loss, loss)


# speedup vs baseline: 1.2000x; 1.2000x over previous
import functools

import jax
import jax.numpy as jnp
from jax import lax
from jax.experimental import pallas as pl
from jax.experimental.pallas import tpu as pltpu
from jax.experimental.pallas import tpu_sc as plsc

D_MODEL_ = 2048
N_FEAT_ = 32768
K_ = 64
BATCH_ = 1024

F_TILE = 1024
NW = 32
ROWS_W = BATCH_ // NW
CAP = 256
NV = CAP // 16
CAPL = 32
import numpy as _np
M_SIGN = _np.int32(-2**31)
S_MIN = _np.int32(-2**31)
I_BIG = _np.int32(2**30)


def _encode_body(x_ref, w_ref, be_ref, bd_ref, out_ref, cmax_ref):
    xt = x_ref[...] - bd_ref[...]
    acts = lax.dot_general(
        xt, w_ref[...],
        dimension_numbers=(((1,), (1,)), ((), ())),
        preferred_element_type=jnp.float32,
    ) + be_ref[...]
    out_ref[...] = acts
    j = pl.program_id(0)

    @pl.when(j == 0)
    def _init():
        cmax_ref[...] = acts

    @pl.when(j > 0)
    def _acc():
        cmax_ref[...] = jnp.maximum(cmax_ref[...], acts)


def _encode(x, W_enc, b_enc, b_dec):
    B, D = x.shape
    F = W_enc.shape[0]
    grid = (F // F_TILE,)
    out_shapes = (
        jax.ShapeDtypeStruct((B, F), jnp.float32),
        jax.ShapeDtypeStruct((B, F_TILE), jnp.float32),
    )
    return pl.pallas_call(
        _encode_body,
        grid=grid,
        in_specs=[
            pl.BlockSpec((B, D), lambda j: (0, 0)),
            pl.BlockSpec((F_TILE, D), lambda j: (j, 0)),
            pl.BlockSpec((1, F_TILE), lambda j: (0, j)),
            pl.BlockSpec((1, D), lambda j: (0, 0)),
        ],
        out_specs=(
            pl.BlockSpec((B, F_TILE), lambda j: (0, j)),
            pl.BlockSpec((B, F_TILE), lambda j: (0, 0)),
        ),
        out_shape=out_shapes,
    )(x, W_enc, b_enc.reshape(1, F), b_dec.reshape(1, D))


def _vperm(x, idx):
    dnums = lax.GatherDimensionNumbers(
        offset_dims=(), collapsed_slice_dims=(0,), start_index_map=(0,))
    return lax.gather(x, idx[:, None], dnums, slice_sizes=(1,),
                      mode=lax.GatherScatterMode.PROMISE_IN_BOUNDS)


def _f2s(b):
    return jnp.where(b < 0, jnp.bitwise_xor(jnp.invert(b), M_SIGN), b)


def _s2b(s):
    return jnp.where(s < 0, jnp.bitwise_xor(jnp.invert(s), M_SIGN), s)


def _topk_body(pre_hbm, cmax_hbm, vals_hbm, idx_hbm,
               rowbuf0, rowbuf1, cmaxbuf, pbuf, lanebuf, candbuf, skeys, sidx,
               ovals, oidx, sem_a, sem_b):
    wid = lax.axis_index("s") * 2 + lax.axis_index("c")
    base = wid * ROWS_W
    lane = lax.iota(jnp.int32, 16)
    lane0 = lane == 0

    pltpu.sync_copy(cmax_hbm.at[pl.ds(base, ROWS_W)], cmaxbuf)
    pltpu.make_async_copy(pre_hbm.at[base], rowbuf0, sem_a).start()

    def process(row_ref, rl):
        def prep(i, _):
            v = cmaxbuf[rl, pl.ds(i * 16, 16)]
            s = _f2s(lax.bitcast_convert_type(v, jnp.int32))
            pbuf[pl.ds(i * 16, 16)] = (s >> 20) + 2048
            return 0
        lax.fori_loop(0, 64, prep, 0, unroll=8)

        def probe(it, p):
            tryp = p + jnp.full((16,), 1 << (11 - it), jnp.int32)
            acc = jnp.zeros(16, jnp.int32)
            for i in range(64):
                m = pbuf[pl.ds(i * 16, 16)] >= tryp
                acc = acc + jnp.where(m, 1, 0)
            for sh in (1, 2, 4, 8):
                acc = acc + _vperm(acc, jnp.bitwise_xor(lane, sh))
            return jnp.where(acc >= K_, tryp, p)
        p = lax.fori_loop(0, 12, probe, jnp.zeros(16, jnp.int32))
        st = (p - 2048) << 20
        tf = lax.bitcast_convert_type(_s2b(st), jnp.float32)

        lbase0 = lane * CAPL
        def extract(i, carry):
            loff, idxv = carry
            for j in range(8):
                v = row_ref[pl.ds((i * 8 + j) * 16, 16)]
                m = v >= tf
                pos = lbase0 + jnp.minimum(loff, CAPL - 1)
                plsc.store_scatter(lanebuf, [pos], idxv, mask=m)
                loff = loff + jnp.where(m, 1, 0)
                idxv = idxv + 16
            return loff, idxv
        loff, _ = lax.fori_loop(0, 256, extract,
                                (jnp.zeros(16, jnp.int32), lane))
        loff = jnp.minimum(loff, CAPL)

        ps = loff
        for sh in (1, 2, 4, 8):
            ps = ps + _vperm(ps, jnp.bitwise_xor(lane, sh))
        incl = loff
        for sh in (1, 2, 4, 8):
            shifted = _vperm(incl, jnp.maximum(lane - sh, 0))
            incl = incl + jnp.where(lane >= sh, shifted, 0)
        lbase = incl - loff
        cvalid = jnp.minimum(ps, CAP)

        for j in range(2 * 16):
            l, half = j // 2, j % 2
            cj = lanebuf[pl.ds(j * 16, 16)]
            lb = _vperm(lbase, jnp.full((16,), l, jnp.int32))
            cl = _vperm(loff, jnp.full((16,), l, jnp.int32))
            slot = lane + 16 * half
            valid = slot < cl
            pos = jnp.minimum(lb + slot, CAP + 15)
            plsc.store_scatter(candbuf, [pos], cj, mask=valid)

        for i in range(NV):
            ci = candbuf[pl.ds(i * 16, 16)]
            valid = (lane + i * 16) < cvalid
            ci = jnp.where(valid, ci, 0)
            v = plsc.load_gather(row_ref, [ci])
            s = _f2s(lax.bitcast_convert_type(v, jnp.int32))
            s = jnp.where(valid, s, S_MIN)
            ci = jnp.where(valid, ci, I_BIG)
            sk, sv = plsc.sort_key_val(s, ci, descending=True)
            skeys[pl.ds(i * 17, 16)] = sk
            sidx[pl.ds(i * 17, 16)] = sv
            skeys[pl.ds(i * 17 + 16, 16)] = jnp.full((16,), S_MIN, jnp.int32)
            sidx[pl.ds(i * 17 + 16, 16)] = jnp.full((16,), I_BIG, jnp.int32)

        rlv = jnp.full((16,), rl, jnp.int32)
        zlane = jnp.zeros(16, jnp.int32)
        def pick(t, hp):
            hs = plsc.load_gather(skeys, [hp])
            hi = plsc.load_gather(sidx, [hp])
            ss, si = plsc.sort_key_val(hs, hi, descending=True)
            mx = _vperm(ss, zlane)
            iw = _vperm(si, zlane)
            win = (hs == mx) & (hi == iw)
            hp = hp + jnp.where(win, 1, 0)
            fv = jnp.maximum(lax.bitcast_convert_type(_s2b(mx), jnp.float32), 0.0)
            posv = jnp.full((16,), t, jnp.int32)
            plsc.store_scatter(ovals, [rlv, posv], fv, mask=lane0)
            plsc.store_scatter(oidx, [rlv, posv], iw, mask=lane0)
            return hp
        lax.fori_loop(0, K_, pick, lane * 17)

    def outer(k, _):
        r0 = base + 2 * k
        pltpu.make_async_copy(pre_hbm.at[r0 + 1], rowbuf1, sem_b).start()
        pltpu.make_async_copy(pre_hbm.at[r0], rowbuf0, sem_a).wait()
        process(rowbuf0, 2 * k)

        @pl.when(k < ROWS_W // 2 - 1)
        def _n0():
            pltpu.make_async_copy(pre_hbm.at[r0 + 2], rowbuf0, sem_a).start()
        pltpu.make_async_copy(pre_hbm.at[r0 + 1], rowbuf1, sem_b).wait()
        process(rowbuf1, 2 * k + 1)

        @pl.when(k < ROWS_W // 2 - 1)
        def _n1():
            pltpu.make_async_copy(pre_hbm.at[r0 + 3], rowbuf1, sem_b).start()
        return 0
    lax.fori_loop(0, ROWS_W // 2, outer, 0)

    pltpu.sync_copy(ovals, vals_hbm.at[pl.ds(base, ROWS_W)])
    pltpu.sync_copy(oidx, idx_hbm.at[pl.ds(base, ROWS_W)])


def _topk_sc(pre_acts, cmax):
    B = pre_acts.shape[0]
    kfn = pl.kernel(
        _topk_body,
        out_type=(
            jax.ShapeDtypeStruct((B, K_), jnp.float32),
            jax.ShapeDtypeStruct((B, K_), jnp.int32),
        ),
        mesh=plsc.VectorSubcoreMesh(core_axis_name="c", subcore_axis_name="s"),
        compiler_params=pltpu.CompilerParams(needs_layout_passes=False),
        scratch_types=[
            pltpu.VMEM((N_FEAT_,), jnp.float32),
            pltpu.VMEM((N_FEAT_,), jnp.float32),
            pltpu.VMEM((ROWS_W, F_TILE), jnp.float32),
            pltpu.VMEM((F_TILE,), jnp.int32),
            pltpu.VMEM((16 * CAPL,), jnp.int32),
            pltpu.VMEM((CAP + 32,), jnp.int32),
            pltpu.VMEM((NV * 17 + 16,), jnp.int32),
            pltpu.VMEM((NV * 17 + 16,), jnp.int32),
            pltpu.VMEM((ROWS_W, K_), jnp.float32),
            pltpu.VMEM((ROWS_W, K_), jnp.int32),
            pltpu.SemaphoreType.DMA,
            pltpu.SemaphoreType.DMA,
        ],
    )
    return kfn(pre_acts, cmax)


SEGS = D_MODEL_ // 256


def _decode_body(wdec_hbm, vals_hbm, idx_hbm, x_hbm, bd_hbm,
                 xhat_hbm, part_hbm,
                 idxb, valb, bdec, acc,
                 g0, g1, xb0, xb1, xh0, xh1, lsbuf,
                 sem_g0, sem_g1, sem_x0, sem_x1, sem_o0, sem_o1):
    wid = lax.axis_index("s") * 2 + lax.axis_index("c")
    base = wid * ROWS_W

    pltpu.sync_copy(idx_hbm.at[pl.ds(base, ROWS_W)], idxb)
    pltpu.sync_copy(vals_hbm.at[pl.ds(base, ROWS_W)], valb)
    pltpu.sync_copy(bd_hbm, bdec)
    pltpu.make_async_copy(x_hbm.at[base], xb0, sem_x0).start()

    def gslice(r, kc):
        return idxb.at[r, pl.ds(kc * 16, 16)]

    def process(r, rl, xb, xh, sem_x, sem_o, lsum):
        pltpu.make_async_copy(wdec_hbm.at[gslice(rl, 0)], g0, sem_g0).start()

        for kc in range(4):
            gb, sem = (g0, sem_g0) if kc % 2 == 0 else (g1, sem_g1)
            nb, nsem = (g1, sem_g1) if kc % 2 == 0 else (g0, sem_g0)
            if kc < 3:
                pltpu.make_async_copy(wdec_hbm.at[gslice(rl, kc + 1)],
                                      nb, nsem).start()
            pltpu.make_async_copy(wdec_hbm.at[gslice(rl, kc)], gb, sem).wait()
            vals16 = valb[rl, pl.ds(kc * 16, 16)]

            def seg_body(sg, _):
                sb = sg * 256
                if kc == 0:
                    a = [jnp.zeros(16, jnp.float32) for _ in range(16)]
                else:
                    a = [acc[pl.ds(sb + j * 16, 16)] for j in range(16)]
                for k in range(16):
                    vv = jnp.full((16,), vals16[k], jnp.float32)
                    for j in range(16):
                        a[j] = a[j] + vv * gb[k, pl.ds(sb + j * 16, 16)]
                for j in range(16):
                    acc[pl.ds(sb + j * 16, 16)] = a[j]
                return _
            lax.fori_loop(0, SEGS, seg_body, 0)

        pltpu.make_async_copy(x_hbm.at[r], xb, sem_x).wait()

        def fin_body(i, ls):
            sl = pl.ds(i * 16, 16)
            xh_v = acc[sl] + bdec[sl]
            xh[sl] = xh_v
            d = xh_v - xb[sl]
            return ls + d * d
        lsum = lax.fori_loop(0, D_MODEL_ // 16, fin_body, lsum, unroll=8)
        pltpu.make_async_copy(xh, xhat_hbm.at[r], sem_o).start()
        return lsum

    def outer(k, lsum):
        r0 = base + 2 * k

        @pl.when(k == 0)
        def _p1():
            pltpu.make_async_copy(x_hbm.at[r0 + 1], xb1, sem_x1).start()
        lsum = process(r0, 2 * k, xb0, xh0, sem_x0, sem_o0, lsum)

        @pl.when(k < ROWS_W // 2 - 1)
        def _n0():
            pltpu.make_async_copy(x_hbm.at[r0 + 2], xb0, sem_x0).start()
        lsum = process(r0 + 1, 2 * k + 1, xb1, xh1, sem_x1, sem_o1, lsum)

        @pl.when(k < ROWS_W // 2 - 1)
        def _n1():
            pltpu.make_async_copy(x_hbm.at[r0 + 3], xb1, sem_x1).start()
        pltpu.make_async_copy(xh0, xhat_hbm.at[r0], sem_o0).wait()
        pltpu.make_async_copy(xh1, xhat_hbm.at[r0 + 1], sem_o1).wait()
        return lsum

    lsum = lax.fori_loop(0, ROWS_W // 2, outer, jnp.zeros(16, jnp.float32))
    lsbuf[...] = lsum
    pltpu.sync_copy(lsbuf, part_hbm.at[wid])


def _decode_sc(W_dec, topk_vals, topk_idx, x, b_dec):
    B = x.shape[0]
    D = x.shape[1]
    kfn = pl.kernel(
        _decode_body,
        out_type=(
            jax.ShapeDtypeStruct((B, D), jnp.float32),
            jax.ShapeDtypeStruct((NW, 16), jnp.float32),
        ),
        mesh=plsc.VectorSubcoreMesh(core_axis_name="c", subcore_axis_name="s"),
        compiler_params=pltpu.CompilerParams(needs_layout_passes=False),
        scratch_types=[
            pltpu.VMEM((ROWS_W, K_), jnp.int32),
            pltpu.VMEM((ROWS_W, K_), jnp.float32),
            pltpu.VMEM((D_MODEL_,), jnp.float32),
            pltpu.VMEM((D_MODEL_,), jnp.float32),
            pltpu.VMEM((16, D_MODEL_), jnp.float32),
            pltpu.VMEM((16, D_MODEL_), jnp.float32),
            pltpu.VMEM((D_MODEL_,), jnp.float32),
            pltpu.VMEM((D_MODEL_,), jnp.float32),
            pltpu.VMEM((D_MODEL_,), jnp.float32),
            pltpu.VMEM((D_MODEL_,), jnp.float32),
            pltpu.VMEM((16,), jnp.float32),
            pltpu.SemaphoreType.DMA,
            pltpu.SemaphoreType.DMA,
            pltpu.SemaphoreType.DMA,
            pltpu.SemaphoreType.DMA,
            pltpu.SemaphoreType.DMA,
            pltpu.SemaphoreType.DMA,
        ],
    )
    return kfn(W_dec, topk_vals, topk_idx, x, b_dec)


def kernel(x, W_enc, b_enc, W_dec, b_dec):
    B = x.shape[0]
    F = W_enc.shape[0]
    pre_acts, cmax = _encode(x, W_enc, b_enc, b_dec)
    topk_vals, topk_idx = _topk_sc(pre_acts, cmax)
    x_hat, partials = _decode_sc(W_dec, topk_vals, topk_idx, x, b_dec)
    recon_loss = jnp.sum(partials) / (x.shape[0] * x.shape[1])
    aux_loss = jnp.float32(0.0)
    loss = recon_loss
    return (x_hat, topk_vals, topk_idx, recon_loss, aux_loss, loss)
